# Initial kernel scaffold; baseline (speedup 1.0000x reference)
import functools

import jax
import jax.numpy as jnp
from jax import lax
from jax.experimental import pallas as pl
from jax.experimental.pallas import tpu as pltpu
from jax.experimental.pallas import tpu_sc as plsc

MAXLEN = 200
EMBED = 32
BATCH = 4096
ROWS = BATCH * MAXLEN

NC, NS = 2, 16
NW = NC * NS
RPW = ROWS // NW

CHUNK = 800
NCHUNK = RPW // CHUNK
IDX_MINOR = 100
IDX_ROWS = CHUNK // IDX_MINOR


def _body(val_ref, tok_ref, pat_ref, out_ref, idx_v, buf_v, pat_v, sem):
    wid = lax.axis_index("s") * NC + lax.axis_index("c")
    pltpu.sync_copy(pat_ref, pat_v)
    base_i = wid * (RPW // IDX_MINOR)
    base_o = wid * RPW

    def chunk_body(c, carry):
        r0 = base_i + c * IDX_ROWS
        g0 = base_o + c * CHUNK
        pltpu.sync_copy(val_ref.at[pl.ds(r0, IDX_ROWS)], idx_v)
        copies = []
        for j in range(IDX_ROWS):
            copies.append(
                pltpu.async_copy(
                    tok_ref.at[idx_v.at[j]],
                    buf_v.at[pl.ds(j * IDX_MINOR, IDX_MINOR)],
                    sem,
                )
            )
        for cp in copies:
            cp.wait()

        def add_step(r, acc):
            for h in range(EMBED // 16):
                sl = pl.ds(h * 16, 16)
                plsc.addupdate(buf_v.at[r, sl], pat_v[r, sl])
            return acc

        lax.fori_loop(0, CHUNK, add_step, 0)
        pltpu.sync_copy(buf_v, out_ref.at[pl.ds(g0, CHUNK)])
        return carry

    lax.fori_loop(0, NCHUNK, chunk_body, 0)


@jax.jit
def _run(val2d, token_table, pattern):
    mesh = plsc.VectorSubcoreMesh(core_axis_name="c", subcore_axis_name="s")
    k = functools.partial(
        pl.kernel,
        mesh=mesh,
        out_type=jax.ShapeDtypeStruct((ROWS, EMBED), jnp.float32),
        scratch_types=[
            pltpu.VMEM((IDX_ROWS, IDX_MINOR), jnp.int32),
            pltpu.VMEM((CHUNK, EMBED), jnp.float32),
            pltpu.VMEM((CHUNK, EMBED), jnp.float32),
            pltpu.SemaphoreType.DMA,
        ],
    )(_body)
    return k(val2d, token_table, pattern)


def kernel(val, token_table, pos_table):
    val2d = val.reshape(ROWS // IDX_MINOR, IDX_MINOR).astype(jnp.int32)
    pattern = jnp.tile(pos_table, (CHUNK // MAXLEN, 1))
    out = _run(val2d, token_table, pattern)
    return out.reshape(BATCH, MAXLEN, EMBED)


# speedup vs baseline: 1.3247x; 1.3247x over previous
import functools

import jax
import jax.numpy as jnp
from jax import lax
from jax.experimental import pallas as pl
from jax.experimental.pallas import tpu as pltpu
from jax.experimental.pallas import tpu_sc as plsc

MAXLEN = 200
EMBED = 32
BATCH = 4096
ROWS = BATCH * MAXLEN

NC, NS = 2, 16
NW = NC * NS
RPW = ROWS // NW

CHUNK = 800
NCHUNK = RPW // CHUNK
IDX_MINOR = 100
IDX_ROWS = CHUNK // IDX_MINOR


def _body(val_ref, tok_ref, pat_ref, out_ref, idx_v, buf_v, pat_v, sem):
    wid = lax.axis_index("s") * NC + lax.axis_index("c")
    pltpu.sync_copy(pat_ref, pat_v)
    base_i = wid * (RPW // IDX_MINOR)
    base_o = wid * RPW

    def chunk_body(c, carry):
        r0 = base_i + c * IDX_ROWS
        g0 = base_o + c * CHUNK
        pltpu.sync_copy(val_ref.at[pl.ds(r0, IDX_ROWS)], idx_v)
        copies = []
        for j in range(IDX_ROWS):
            copies.append(
                pltpu.async_copy(
                    tok_ref.at[idx_v.at[j]],
                    buf_v.at[pl.ds(j * IDX_MINOR, IDX_MINOR)],
                    sem,
                )
            )
        for cp in copies:
            cp.wait()

        def add_step(r, acc):
            for h in range(EMBED // 16):
                sl = pl.ds(h * 16, 16)
                plsc.addupdate(buf_v.at[r, sl], pat_v[r, sl])
            return acc

        lax.fori_loop(0, CHUNK, add_step, 0)
        pltpu.sync_copy(buf_v, out_ref.at[pl.ds(g0, CHUNK)])
        return carry

    lax.fori_loop(0, NCHUNK, chunk_body, 0)


@jax.jit
def _run(val2d, token_table, pattern):
    mesh = plsc.VectorSubcoreMesh(core_axis_name="c", subcore_axis_name="s")
    k = functools.partial(
        pl.kernel,
        mesh=mesh,
        out_type=jax.ShapeDtypeStruct((ROWS, EMBED), jnp.float32),
        scratch_types=[
            pltpu.VMEM((IDX_ROWS, IDX_MINOR), jnp.int32),
            pltpu.VMEM((CHUNK, EMBED), jnp.float32),
            pltpu.VMEM((CHUNK, EMBED), jnp.float32),
            pltpu.SemaphoreType.DMA,
        ],
        compiler_params=pltpu.CompilerParams(use_tc_tiling_on_sc=False),
    )(_body)
    return k(val2d, token_table, pattern)


def kernel(val, token_table, pos_table):
    val2d = val.reshape(ROWS // IDX_MINOR, IDX_MINOR).astype(jnp.int32)
    pattern = jnp.tile(pos_table, (CHUNK // MAXLEN, 1))
    out = _run(val2d, token_table, pattern)
    return out.reshape(BATCH, MAXLEN, EMBED)
